# baseline (speedup 1.0000x reference)
import jax
import jax.numpy as jnp
from jax import lax
from jax.experimental import pallas as pl
from jax.experimental.pallas import tpu as pltpu

_BH = 16
_NBUF = 4


def _crop_body(rois_ref, data_hbm, out_hbm, inb, outb, insem, outsem):
    nch = data_hbm.shape[0] // _BH
    w = data_hbm.shape[1]
    n = data_hbm.shape[2]

    def in_copy(c, b):
        return pltpu.make_async_copy(
            data_hbm.at[pl.ds(c * _BH, _BH)], inb.at[b], insem.at[b])

    def out_copy(c, b):
        return pltpu.make_async_copy(
            outb.at[b], out_hbm.at[pl.ds(c * _BH, _BH)], outsem.at[b])

    for b in range(_NBUF):
        in_copy(b, b).start()

    x1 = rois_ref[0, :][None, None, :]
    y1 = rois_ref[1, :][None, None, :]
    x2 = rois_ref[2, :][None, None, :]
    y2 = rois_ref[3, :][None, None, :]

    def superstep(s, carry):
        for b in range(_NBUF):
            c = s * _NBUF + b
            in_copy(c, b).wait()

            @pl.when(s >= 1)
            def _():
                out_copy(c - _NBUF, b).wait()

            ww = lax.broadcasted_iota(jnp.int32, (1, w, 1), 1).astype(jnp.float32)
            hh = (lax.broadcasted_iota(jnp.int32, (_BH, 1, 1), 0).astype(jnp.float32)
                  + (c * _BH).astype(jnp.float32))
            xm = (ww >= x1) & (ww <= x2)
            ym = (hh >= y1) & (hh <= y2)
            outb[b] = jnp.where(xm & ym, inb[b], 0.0)

            out_copy(c, b).start()

            @pl.when(c + _NBUF < nch)
            def _():
                in_copy(c + _NBUF, b).start()
        return carry

    lax.fori_loop(0, nch // _NBUF, superstep, 0)

    for b in range(_NBUF):
        out_copy(nch - _NBUF + b, b).wait()


def kernel(data, rois):
    h, w, n = data.shape
    rois_t = rois.T
    return pl.pallas_call(
        _crop_body,
        in_specs=[
            pl.BlockSpec(memory_space=pltpu.MemorySpace.VMEM),
            pl.BlockSpec(memory_space=pltpu.MemorySpace.HBM),
        ],
        out_specs=pl.BlockSpec(memory_space=pltpu.MemorySpace.HBM),
        out_shape=jax.ShapeDtypeStruct((h, w, n), data.dtype),
        scratch_shapes=[
            pltpu.VMEM((_NBUF, _BH, w, n), jnp.float32),
            pltpu.VMEM((_NBUF, _BH, w, n), jnp.float32),
            pltpu.SemaphoreType.DMA((_NBUF,)),
            pltpu.SemaphoreType.DMA((_NBUF,)),
        ],
        compiler_params=pltpu.CompilerParams(
            vmem_limit_bytes=55 * 1024 * 1024,
        ),
    )(rois_t, data)
